# baseline (speedup 1.0000x reference)
import jax
import jax.numpy as jnp
from jax import lax
from jax.experimental import pallas as pl
from jax.experimental.pallas import tpu as pltpu
from jax.experimental.pallas import tpu_sc as plsc

_NC, _NS = 2, 16
_NW = _NC * _NS
_G = 16
_D = 16
_W = 128


def _conds(tvec):
    ts = [tvec[k] for k in range(_G)]
    conds = [None] * _G
    slots = [None] * _G
    nfire = None
    for k in range(_G):
        if k == 0:
            conds[0] = None
            slots[0] = jnp.int32(0)
            nfire = jnp.int32(1)
        else:
            c = ts[k] != ts[k - 1]
            conds[k] = c
            slots[k] = jnp.where(c, jnp.int32(k), slots[k - 1])
            nfire = nfire + c.astype(jnp.int32)
    return ts, conds, slots, nfire


def _fire(tableT, gvec, ring, sem):
    tvec = lax.shift_right_logical(gvec, 7)
    ts, conds, _, _ = _conds(tvec)
    for k in range(_G):
        t = pl.multiple_of(ts[k] * _W, _W)
        if conds[k] is None:
            pltpu.async_copy(tableT.at[:, pl.ds(t, _W)], ring.at[k], sem)
        else:
            @pl.when(conds[k])
            def _(t=t, k=k):
                pltpu.async_copy(tableT.at[:, pl.ds(t, _W)], ring.at[k], sem)


def _process(tableT, gvec, ebase, ring, sem, stag_v, lanes):
    tvec = lax.shift_right_logical(gvec, 7)
    _, conds, slots, nfire = _conds(tvec)

    def drain(i, c):
        pltpu.make_async_copy(
            tableT.at[:, pl.ds(0, _W)], ring.at[0], sem
        ).wait()
        return c

    lax.fori_loop(0, nfire, drain, 0)

    lvec = gvec & (_W - 1)
    for k in range(_G):
        l = lvec[k]
        lp = jnp.minimum(l, _W - _G)
        dvec = jnp.full((_G,), 0, jnp.int32) + (l - lp)
        acc = jnp.full((_G,), 0.0, jnp.float32)
        s = slots[k]
        for c in range(_D):
            sub = ring[s, c, pl.ds(lp, _G)]
            w = jnp.take(sub, dvec)
            acc = jnp.where(lanes == c, w, acc)
        stag_v[pl.ds((ebase + k) * _D, _D)] = acc


def _body(tableT, ids_hbm, ord_hbm, out_hbm, ids_v, ord_v, ring_a, ring_b,
          stag_v, offs_v, sem_a, sem_b, sem_s):
    wid = lax.axis_index("s") * _NC + lax.axis_index("c")
    bpw = ids_v.shape[0]
    base = wid * bpw
    ngrp = bpw // _G
    pltpu.sync_copy(ids_hbm.at[pl.ds(base, bpw)], ids_v)
    pltpu.sync_copy(ord_hbm.at[pl.ds(base, bpw)], ord_v)
    lanes = lax.iota(jnp.int32, _G)

    def offs(i, c):
        ovec = ord_v[pl.ds(i * _G, _G)]
        for k in range(_G):
            off16 = jnp.full((_G,), 0, jnp.int32) + ovec[k] * _D + lanes
            offs_v[pl.ds((i * _G + k) * _D, _D)] = off16
        return c

    lax.fori_loop(0, ngrp, offs, 0)

    def group(g):
        return ids_v[pl.ds(g * _G, _G)]

    _fire(tableT, group(0), ring_a, sem_a)

    def step(i, carry):
        ga = 2 * i
        gb = 2 * i + 1
        _fire(tableT, group(gb), ring_b, sem_b)
        _process(tableT, group(ga), ga * _G, ring_a, sem_a, stag_v, lanes)

        @pl.when(i < ngrp // 2 - 1)
        def _():
            _fire(tableT, group(ga + 2), ring_a, sem_a)

        _process(tableT, group(gb), gb * _G, ring_b, sem_b, stag_v, lanes)
        return carry

    lax.fori_loop(0, ngrp // 2, step, 0)
    pltpu.async_copy(stag_v, out_hbm.at[offs_v], sem_s).wait()


def kernel(preds, idx):
    B = idx.shape[0]
    D = preds.shape[1]
    bpw = B // _NW
    tableT = preds.T
    idx32 = idx.astype(jnp.int32)
    order = jnp.argsort(idx32).astype(jnp.int32)
    ids = jnp.sort(idx32)
    mesh = plsc.VectorSubcoreMesh(core_axis_name="c", subcore_axis_name="s")
    out = pl.kernel(
        _body,
        out_type=jax.ShapeDtypeStruct((B * D,), jnp.float32),
        mesh=mesh,
        scratch_types=[
            pltpu.VMEM((bpw,), jnp.int32),
            pltpu.VMEM((bpw,), jnp.int32),
            pltpu.VMEM((_G, _D, _W), jnp.float32),
            pltpu.VMEM((_G, _D, _W), jnp.float32),
            pltpu.VMEM((bpw * _D,), jnp.float32),
            pltpu.VMEM((bpw * _D,), jnp.int32),
            pltpu.SemaphoreType.DMA,
            pltpu.SemaphoreType.DMA,
            pltpu.SemaphoreType.DMA,
        ],
    )(tableT, ids, order)
    return out.reshape(B, D)


# speedup vs baseline: 7.2146x; 7.2146x over previous
import jax
import jax.numpy as jnp
from jax import lax
from jax.experimental import pallas as pl
from jax.experimental.pallas import tpu as pltpu
from jax.experimental.pallas import tpu_sc as plsc

_NC, _NS = 2, 16
_NW = _NC * _NS
_G = 16
_D = 16
_W = 128


def _fire(tableT, g_vec, ring, sem):
    for k in range(_G):
        t = pl.multiple_of(g_vec[k] & -_W, _W)
        pltpu.async_copy(tableT.at[:, pl.ds(t, _W)], ring.at[k], sem)


def _process(tableT, g_vec, jb, ring, sem, out_v, lanes):
    for k in range(_G):
        pltpu.make_async_copy(
            tableT.at[:, pl.ds(0, _W)], ring.at[k], sem
        ).wait()
    lvec = g_vec & (_W - 1)
    dvecs = []
    lps = []
    for k in range(_G):
        l = lvec[k]
        lp = jnp.minimum(l, _W - _G)
        lps.append(lp)
        dvecs.append(jnp.full((_G,), 0, jnp.int32) + (l - lp))
    for c in range(_D):
        acc = jnp.full((_G,), 0.0, jnp.float32)
        for k in range(_G):
            sub = ring[k, c, pl.ds(lps[k], _G)]
            w = jnp.take(sub, dvecs[k])
            acc = jnp.where(lanes == k, w, acc)
        out_v[c, pl.ds(jb, _G)] = acc


def _body(tableT, idx_hbm, out_hbm, idx_v, ring_a, ring_b, out_v,
          sem_a, sem_b):
    wid = lax.axis_index("s") * _NC + lax.axis_index("c")
    bpw = idx_v.shape[0]
    base = wid * bpw
    ngrp = bpw // _G
    pltpu.sync_copy(idx_hbm.at[pl.ds(base, bpw)], idx_v)
    lanes = lax.iota(jnp.int32, _G)

    def group(g):
        return idx_v[pl.ds(g * _G, _G)]

    _fire(tableT, group(0), ring_a, sem_a)

    def step(i, carry):
        ga = 2 * i
        gb = 2 * i + 1
        _fire(tableT, group(gb), ring_b, sem_b)
        _process(tableT, group(ga), ga * _G, ring_a, sem_a, out_v, lanes)

        @pl.when(i < ngrp // 2 - 1)
        def _():
            _fire(tableT, group(ga + 2), ring_a, sem_a)

        _process(tableT, group(gb), gb * _G, ring_b, sem_b, out_v, lanes)
        return carry

    lax.fori_loop(0, ngrp // 2, step, 0)
    pltpu.sync_copy(out_v, out_hbm.at[:, pl.ds(base, bpw)])


def kernel(preds, idx):
    B = idx.shape[0]
    D = preds.shape[1]
    bpw = B // _NW
    tableT = preds.T
    idx32 = idx.astype(jnp.int32)
    mesh = plsc.VectorSubcoreMesh(core_axis_name="c", subcore_axis_name="s")
    out = pl.kernel(
        _body,
        out_type=jax.ShapeDtypeStruct((D, B), jnp.float32),
        mesh=mesh,
        scratch_types=[
            pltpu.VMEM((bpw,), jnp.int32),
            pltpu.VMEM((_G, _D, _W), jnp.float32),
            pltpu.VMEM((_G, _D, _W), jnp.float32),
            pltpu.VMEM((D, bpw), jnp.float32),
            pltpu.SemaphoreType.DMA,
            pltpu.SemaphoreType.DMA,
        ],
    )(tableT, idx32)
    return out.T
